# Initial kernel scaffold; baseline (speedup 1.0000x reference)
import jax
import jax.numpy as jnp
from jax.experimental import pallas as pl


def kernel(x, weight):
    raise NotImplementedError("write your pallas kernel here")


# speedup vs baseline: 1.5623x; 1.5623x over previous
import functools

import jax
import jax.numpy as jnp
from jax.experimental import pallas as pl

_NUM_EXPERTS = 64
_TOP_K = 8
_BT = 512


def _router_block(x_ref, w_ref, wout_ref, iout_ref):
    s = jax.lax.dot_general(
        x_ref[...], w_ref[...],
        dimension_numbers=(((1,), (1,)), ((), ())),
        preferred_element_type=jnp.float32,
    )
    m = jnp.max(s, axis=-1, keepdims=True)
    e = jnp.exp(s - m)
    p = e / jnp.sum(e, axis=-1, keepdims=True)

    iota = jax.lax.broadcasted_iota(jnp.int32, p.shape, 1)
    vals = []
    idxs = []
    work = p
    for _ in range(_TOP_K):
        mx = jnp.max(work, axis=-1, keepdims=True)
        idx = jnp.min(jnp.where(work == mx, iota, _NUM_EXPERTS),
                      axis=-1, keepdims=True)
        vals.append(mx)
        idxs.append(idx)
        work = jnp.where(iota == idx, -jnp.inf, work)
    wout_ref[...] = jnp.concatenate(vals, axis=1)
    iout_ref[...] = jnp.concatenate(idxs, axis=1)


@jax.jit
def kernel(x, weight):
    n_tokens, _ = x.shape
    grid = (n_tokens // _BT,)
    wout, iout = pl.pallas_call(
        _router_block,
        grid=grid,
        in_specs=[
            pl.BlockSpec((_BT, x.shape[1]), lambda i: (i, 0)),
            pl.BlockSpec(weight.shape, lambda i: (0, 0)),
        ],
        out_specs=[
            pl.BlockSpec((_BT, _TOP_K), lambda i: (i, 0)),
            pl.BlockSpec((_BT, _TOP_K), lambda i: (i, 0)),
        ],
        out_shape=[
            jax.ShapeDtypeStruct((n_tokens, _TOP_K), jnp.float32),
            jax.ShapeDtypeStruct((n_tokens, _TOP_K), jnp.int32),
        ],
    )(x, weight)
    return wout, iout


# speedup vs baseline: 1.7950x; 1.1489x over previous
import functools

import jax
import jax.numpy as jnp
from jax.experimental import pallas as pl

_NUM_EXPERTS = 64
_TOP_K = 8
_BT = 1024


def _router_block(x_ref, w_ref, wout_ref, iout_ref):
    s = jax.lax.dot_general(
        x_ref[...], w_ref[...],
        dimension_numbers=(((1,), (1,)), ((), ())),
        preferred_element_type=jnp.float32,
    )
    m = jnp.max(s, axis=-1, keepdims=True)
    e = jnp.exp(s - m)
    p = e / jnp.sum(e, axis=-1, keepdims=True)

    iota = jax.lax.broadcasted_iota(jnp.int32, p.shape, 1)
    vals = []
    idxs = []
    work = p
    for _ in range(_TOP_K):
        mx = jnp.max(work, axis=-1, keepdims=True)
        idx = jnp.min(jnp.where(work == mx, iota, _NUM_EXPERTS),
                      axis=-1, keepdims=True)
        vals.append(mx)
        idxs.append(idx)
        work = jnp.where(iota == idx, -jnp.inf, work)
    wout_ref[...] = jnp.concatenate(vals, axis=1)
    iout_ref[...] = jnp.concatenate(idxs, axis=1)


@jax.jit
def kernel(x, weight):
    n_tokens, _ = x.shape
    grid = (n_tokens // _BT,)
    wout, iout = pl.pallas_call(
        _router_block,
        grid=grid,
        in_specs=[
            pl.BlockSpec((_BT, x.shape[1]), lambda i: (i, 0)),
            pl.BlockSpec(weight.shape, lambda i: (0, 0)),
        ],
        out_specs=[
            pl.BlockSpec((_BT, _TOP_K), lambda i: (i, 0)),
            pl.BlockSpec((_BT, _TOP_K), lambda i: (i, 0)),
        ],
        out_shape=[
            jax.ShapeDtypeStruct((n_tokens, _TOP_K), jnp.float32),
            jax.ShapeDtypeStruct((n_tokens, _TOP_K), jnp.int32),
        ],
    )(x, weight)
    return wout, iout
